# baseline (speedup 1.0000x reference)
import functools

import jax
import jax.numpy as jnp
from jax.experimental import pallas as pl
from jax.experimental.pallas import tpu as pltpu

_BN_EPS = 1e-5
_VMEM_LIMIT = 64 * 1024 * 1024


def _fused_kernel(x_ref, w1_ref, w2_ref, gb_ref, o_ref,
                  h1_scr, s1_scr, q1_scr, s2_scr, q2_scr, *, n, k, tile, inv_m):
    p = pl.program_id(0)
    i = pl.program_id(1)
    t = pl.program_id(2)
    first = jnp.logical_and(i == 0, t == 0)

    def fold(s_scr, q_scr, g, b):
        mean = s_scr[...] * inv_m
        var = jnp.maximum(q_scr[...] * inv_m - mean * mean, 0.0)
        sc = g * jax.lax.rsqrt(var + _BN_EPS)
        return sc, b - mean * sc

    @pl.when(p == 0)
    def _():
        @pl.when(first)
        def _():
            s1_scr[...] = jnp.zeros_like(s1_scr)
            q1_scr[...] = jnp.zeros_like(q1_scr)
        h = jnp.dot(w1_ref[...], x_ref[...], preferred_element_type=jnp.float32)
        s1_scr[...] += jnp.sum(h, axis=1, keepdims=True)
        q1_scr[...] += jnp.sum(h * h, axis=1, keepdims=True)
        h1_scr[i, :, pl.ds(t * tile, tile)] = h.astype(h1_scr.dtype)

    @pl.when(p == 1)
    def _():
        @pl.when(first)
        def _():
            s2_scr[...] = jnp.zeros_like(s2_scr)
            q2_scr[...] = jnp.zeros_like(q2_scr)
        sc1, sh1 = fold(s1_scr, q1_scr, gb_ref[:, 0:1], gb_ref[:, 1:2])
        h1 = h1_scr[i, :, pl.ds(t * tile, tile)].astype(jnp.float32)
        a1 = jnp.maximum(h1 * sc1 + sh1, 0.0)
        h2 = jnp.dot(w2_ref[...], a1, preferred_element_type=jnp.float32)
        s2_scr[...] += jnp.sum(h2, axis=1, keepdims=True)
        q2_scr[...] += jnp.sum(h2 * h2, axis=1, keepdims=True)

    @pl.when(p == 2)
    def _():
        sc1, sh1 = fold(s1_scr, q1_scr, gb_ref[:, 0:1], gb_ref[:, 1:2])
        sc2, sh2 = fold(s2_scr, q2_scr, gb_ref[:, 2:3], gb_ref[:, 3:4])
        h1 = h1_scr[i, :, pl.ds(t * tile, tile)].astype(jnp.float32)
        a1 = jnp.maximum(h1 * sc1 + sh1, 0.0)
        h2 = jnp.dot(w2_ref[...], a1, preferred_element_type=jnp.float32)
        o_ref[...] = jnp.maximum(h2 * sc2 + sh2, 0.0)


def kernel(x, w1, w2, gamma1, beta1, gamma2, beta2):
    n, cin, h, w = x.shape
    cout = w1.shape[0]
    hw = h * w
    inv_m = 1.0 / float(n * hw)

    k = 1
    assert hw % (k * 128) == 0
    tile = hw // k

    x3 = x.reshape(n, cin, hw)
    gb = jnp.stack([gamma1, beta1, gamma2, beta2], axis=1)

    x_spec = pl.BlockSpec(
        (None, cin, tile),
        lambda p, i, t: (jnp.where(p == 0, i, n - 1), 0,
                         jnp.where(p == 0, t, k - 1)))
    o_spec = pl.BlockSpec(
        (None, cout, tile),
        lambda p, i, t: (jnp.where(p == 2, i, 0), 0, jnp.where(p == 2, t, 0)))
    w_spec = lambda a, b: pl.BlockSpec((a, b), lambda p, i, t: (0, 0))

    out = pl.pallas_call(
        functools.partial(_fused_kernel, n=n, k=k, tile=tile, inv_m=inv_m),
        out_shape=jax.ShapeDtypeStruct((n, cout, hw), jnp.float32),
        grid=(3, n, k),
        in_specs=[x_spec, w_spec(cout, cin), w_spec(cout, cout), w_spec(cout, 4)],
        out_specs=o_spec,
        scratch_shapes=[
            pltpu.VMEM((n, cout, hw), jnp.bfloat16),
            pltpu.VMEM((cout, 1), jnp.float32),
            pltpu.VMEM((cout, 1), jnp.float32),
            pltpu.VMEM((cout, 1), jnp.float32),
            pltpu.VMEM((cout, 1), jnp.float32),
        ],
        compiler_params=pltpu.CompilerParams(
            dimension_semantics=("arbitrary", "arbitrary", "arbitrary"),
            vmem_limit_bytes=_VMEM_LIMIT),
    )(x3, w1, w2, gb)

    return out.reshape(n, cout, h, w)


# speedup vs baseline: 2.2994x; 2.2994x over previous
import functools

import jax
import jax.numpy as jnp
from jax.experimental import pallas as pl
from jax.experimental.pallas import tpu as pltpu

_BN_EPS = 1e-5
_VMEM_LIMIT = 64 * 1024 * 1024


def _fused_kernel(x_ref, w1_ref, w2_ref, gb_ref, o_ref,
                  h1_scr, s1_scr, q1_scr, s2_scr, q2_scr,
                  *, n, kc, ht, wd, inv_m):
    p = pl.program_id(0)
    i = pl.program_id(1)
    c = pl.program_id(2)
    first = jnp.logical_and(i == 0, c == 0)
    tile = ht * wd

    def fold(s_scr, q_scr, g, b):
        mean = s_scr[...] * inv_m
        var = jnp.maximum(q_scr[...] * inv_m - mean * mean, 0.0)
        sc = g * jax.lax.rsqrt(var + _BN_EPS)
        return sc, b - mean * sc

    @pl.when(p == 0)
    def _():
        @pl.when(first)
        def _():
            s1_scr[...] = jnp.zeros_like(s1_scr)
            q1_scr[...] = jnp.zeros_like(q1_scr)
        x2 = x_ref[...].astype(jnp.bfloat16).reshape(x_ref.shape[0], tile)
        h = jnp.dot(w1_ref[...].astype(jnp.bfloat16), x2,
                    preferred_element_type=jnp.float32)
        s1_scr[...] += jnp.sum(h, axis=1, keepdims=True)
        q1_scr[...] += jnp.sum(h * h, axis=1, keepdims=True)
        h1_scr[i, :, pl.ds(c * tile, tile)] = h.astype(h1_scr.dtype)

    @pl.when(p == 1)
    def _():
        @pl.when(first)
        def _():
            s2_scr[...] = jnp.zeros_like(s2_scr)
            q2_scr[...] = jnp.zeros_like(q2_scr)
        sc1, sh1 = fold(s1_scr, q1_scr, gb_ref[:, 0:1], gb_ref[:, 1:2])
        h1 = h1_scr[i, :, pl.ds(c * tile, tile)].astype(jnp.float32)
        a1 = jnp.maximum(h1 * sc1 + sh1, 0.0)
        h2 = jnp.dot(w2_ref[...], a1, preferred_element_type=jnp.float32)
        s2_scr[...] += jnp.sum(h2, axis=1, keepdims=True)
        q2_scr[...] += jnp.sum(h2 * h2, axis=1, keepdims=True)

    @pl.when(p == 2)
    def _():
        sc1, sh1 = fold(s1_scr, q1_scr, gb_ref[:, 0:1], gb_ref[:, 1:2])
        sc2, sh2 = fold(s2_scr, q2_scr, gb_ref[:, 2:3], gb_ref[:, 3:4])
        h1 = h1_scr[i, :, pl.ds(c * tile, tile)].astype(jnp.float32)
        a1 = jnp.maximum(h1 * sc1 + sh1, 0.0)
        h2 = jnp.dot(w2_ref[...], a1, preferred_element_type=jnp.float32)
        o = jnp.maximum(h2 * sc2 + sh2, 0.0)
        o_ref[...] = o.reshape(o.shape[0], ht, wd)


def kernel(x, w1, w2, gamma1, beta1, gamma2, beta2):
    n, cin, h, w = x.shape
    cout = w1.shape[0]
    inv_m = 1.0 / float(n * h * w)

    kc = 2
    assert h % kc == 0 and (h // kc) * w % 128 == 0
    ht = h // kc

    gb = jnp.stack([gamma1, beta1, gamma2, beta2], axis=1)

    x_spec = pl.BlockSpec(
        (None, cin, ht, w),
        lambda p, i, c: (jnp.where(p == 0, i, n - 1), 0,
                         jnp.where(p == 0, c, kc - 1), 0))
    o_spec = pl.BlockSpec(
        (None, cout, ht, w),
        lambda p, i, c: (jnp.where(p == 2, i, 0), 0,
                         jnp.where(p == 2, c, 0), 0))
    w_spec = lambda a, b: pl.BlockSpec((a, b), lambda p, i, c: (0, 0))

    out = pl.pallas_call(
        functools.partial(_fused_kernel, n=n, kc=kc, ht=ht, wd=w, inv_m=inv_m),
        out_shape=jax.ShapeDtypeStruct((n, cout, h, w), jnp.float32),
        grid=(3, n, kc),
        in_specs=[x_spec, w_spec(cout, cin), w_spec(cout, cout), w_spec(cout, 4)],
        out_specs=o_spec,
        scratch_shapes=[
            pltpu.VMEM((n, cout, h * w), jnp.bfloat16),
            pltpu.VMEM((cout, 1), jnp.float32),
            pltpu.VMEM((cout, 1), jnp.float32),
            pltpu.VMEM((cout, 1), jnp.float32),
            pltpu.VMEM((cout, 1), jnp.float32),
        ],
        compiler_params=pltpu.CompilerParams(
            dimension_semantics=("arbitrary", "arbitrary", "arbitrary"),
            vmem_limit_bytes=_VMEM_LIMIT),
    )(x, w1, w2, gb)

    return out
